# baseline (device time: 130280 ns/iter reference)
import jax
import jax.numpy as jnp
from jax import lax
from jax.experimental import pallas as pl
from jax.experimental.pallas import tpu as pltpu

T = 4096
D = 1024
CH = 512
N_MAX = T // CH
UNROLL = 8
ROW = (8, 128)
XS_ROWS = T // 2 + 512


def _body(cnt_ref, order_ref, x_ref, out_ref, xs_ref, asm_ref,
          send_sems, recv_sems):
    my_x = lax.axis_index("x")
    my_y = lax.axis_index("y")
    my_z = lax.axis_index("z")
    peer = (my_x, my_y, 1 - my_z)
    cnt0 = cnt_ref[0]

    barrier_sem = pltpu.get_barrier_semaphore()
    pl.semaphore_signal(
        barrier_sem, inc=1, device_id=peer,
        device_id_type=pl.DeviceIdType.MESH,
    )
    pl.semaphore_wait(barrier_sem, 1)

    is0 = my_z == 0
    send_count = jnp.where(is0, T - cnt0, cnt0)
    n_send = (send_count + CH - 1) // CH
    n_keep = (T - send_count + CH - 1) // CH
    dst_shift = jnp.where(is0, -cnt0, T - cnt0)

    def gather_rows(dst_ref, start, dst_base=None):
        def grp(g, _):
            base = start + g * UNROLL
            dbase = (start if dst_base is None else dst_base) + g * UNROLL
            for u in range(UNROLL):
                dst_ref[pl.ds(dbase + u, 1)] = x_ref[
                    pl.ds(order_ref[base + u], 1), :
                ].reshape(1, *ROW)
            return 0

        lax.fori_loop(0, CH // UNROLL, grp, 0)

    def convert_groups(g_lo, g_hi):
        def one(g, _):
            r = pl.multiple_of(g * 8, 8)
            out_ref[pl.ds(r, 8), :] = asm_ref[pl.ds(r, 8)].reshape(8, D)
            return 0

        lax.fori_loop(g_lo, g_hi, one, 0)

    for i in range(N_MAX):
        src_start = jnp.where(
            is0,
            jnp.maximum(T - (i + 1) * CH, cnt0),
            jnp.minimum(i * CH, cnt0 - CH),
        )
        dst_start = src_start + dst_shift

        reg_begin = jnp.where(is0, cnt0, 0)

        @pl.when(i < n_send)
        def _(i=i, src_start=src_start, dst_start=dst_start,
              reg_begin=reg_begin):
            gather_rows(xs_ref, src_start, dst_base=src_start - reg_begin)
            pltpu.make_async_remote_copy(
                src_ref=xs_ref.at[pl.ds(src_start - reg_begin, CH)],
                dst_ref=asm_ref.at[pl.ds(dst_start, CH)],
                send_sem=send_sems.at[i],
                recv_sem=recv_sems.at[i],
                device_id=peer,
                device_id_type=pl.DeviceIdType.MESH,
            ).start()

    for i in range(N_MAX):
        start = jnp.where(
            is0,
            jnp.minimum(i * CH, cnt0 - CH),
            jnp.maximum(T - (i + 1) * CH, cnt0),
        )

        @pl.when(i < n_keep)
        def _(start=start):
            gather_rows(asm_ref, start)

    convert_groups(
        jnp.where(is0, 0, (cnt0 + 7) // 8),
        jnp.where(is0, cnt0 // 8, T // 8),
    )

    for i in range(N_MAX):
        hi_prev = cnt0 + jnp.minimum(i * CH, T - cnt0)
        hi_cur = cnt0 + jnp.minimum((i + 1) * CH, T - cnt0)
        lo_prev = jnp.maximum(cnt0 - i * CH, 0)
        lo_cur = jnp.maximum(cnt0 - (i + 1) * CH, 0)
        g_lo = jnp.where(is0, hi_prev // 8, (lo_cur + 7) // 8)
        g_hi = jnp.where(is0, hi_cur // 8, (lo_prev + 7) // 8)

        @pl.when(i < n_send)
        def _(i=i, g_lo=g_lo, g_hi=g_hi):
            pltpu.make_async_remote_copy(
                src_ref=xs_ref.at[pl.ds(0, CH)],
                dst_ref=asm_ref.at[pl.ds(0, CH)],
                send_sem=send_sems.at[i],
                recv_sem=recv_sems.at[i],
                device_id=peer,
                device_id_type=pl.DeviceIdType.MESH,
            ).wait_recv()
            convert_groups(g_lo, g_hi)

    for i in range(N_MAX):

        @pl.when(i < n_send)
        def _(i=i):
            pltpu.make_async_remote_copy(
                src_ref=xs_ref.at[pl.ds(0, CH)],
                dst_ref=asm_ref.at[pl.ds(0, CH)],
                send_sem=send_sems.at[i],
                recv_sem=recv_sems.at[i],
                device_id=peer,
                device_id_type=pl.DeviceIdType.MESH,
            ).wait_send()


def kernel(x, dest):
    order = jnp.argsort(dest, stable=True).astype(jnp.int32)
    cnt0 = jnp.sum(dest == 0).astype(jnp.int32).reshape((1,))

    return pl.pallas_call(
        _body,
        out_shape=jax.ShapeDtypeStruct((T, D), jnp.float32),
        in_specs=[
            pl.BlockSpec(memory_space=pltpu.SMEM),
            pl.BlockSpec(memory_space=pltpu.SMEM),
            pl.BlockSpec(memory_space=pltpu.VMEM),
        ],
        out_specs=pl.BlockSpec(memory_space=pltpu.VMEM),
        scratch_shapes=[
            pltpu.VMEM((XS_ROWS, *ROW), jnp.float32),
            pltpu.VMEM((T, *ROW), jnp.float32),
            pltpu.SemaphoreType.DMA((N_MAX,)),
            pltpu.SemaphoreType.DMA((N_MAX,)),
        ],
        compiler_params=pltpu.CompilerParams(
            collective_id=0, vmem_limit_bytes=64 * 1024 * 1024
        ),
    )(cnt0, order, x)


# device time: 120074 ns/iter; 1.0850x vs baseline; 1.0850x over previous
import jax
import jax.numpy as jnp
from jax import lax
from jax.experimental import pallas as pl
from jax.experimental.pallas import tpu as pltpu

T = 4096
D = 1024
CH = 512
N_MAX = T // CH
UNROLL = 8
ROW = (8, 128)


def _body(cnt_ref, order_ref, x_ref, out_ref, xs_ref, send_sems, recv_sems):
    my_x = lax.axis_index("x")
    my_y = lax.axis_index("y")
    my_z = lax.axis_index("z")
    peer = (my_x, my_y, 1 - my_z)
    cnt0 = cnt_ref[0]

    barrier_sem = pltpu.get_barrier_semaphore()
    pl.semaphore_signal(
        barrier_sem, inc=1, device_id=peer,
        device_id_type=pl.DeviceIdType.MESH,
    )
    pl.semaphore_wait(barrier_sem, 1)

    is0 = my_z == 0
    send_count = jnp.where(is0, T - cnt0, cnt0)
    keep_count = T - send_count
    n_send = (send_count + CH - 1) // CH
    n_keep = (keep_count + CH - 1) // CH
    dst_shift = jnp.where(is0, -cnt0, T - cnt0)

    def gather_rows(dst_ref, start):
        def grp(g, _):
            base = start + g * UNROLL
            for u in range(UNROLL):
                j = base + u
                dst_ref[pl.ds(j, 1)] = x_ref[
                    pl.ds(order_ref[j], 1), :
                ].reshape(1, *ROW)
            return 0

        lax.fori_loop(0, CH // UNROLL, grp, 0)

    for i in range(N_MAX):
        src_start = jnp.where(
            is0,
            jnp.maximum(T - (i + 1) * CH, cnt0),
            jnp.minimum(i * CH, cnt0 - CH),
        )
        dst_start = src_start + dst_shift

        @pl.when(i < n_send)
        def _(i=i, src_start=src_start, dst_start=dst_start):
            gather_rows(xs_ref, src_start)
            pltpu.make_async_remote_copy(
                src_ref=xs_ref.at[pl.ds(src_start, CH)],
                dst_ref=out_ref.at[pl.ds(dst_start, CH)],
                send_sem=send_sems.at[i],
                recv_sem=recv_sems.at[i],
                device_id=peer,
                device_id_type=pl.DeviceIdType.MESH,
            ).start()

    for i in range(N_MAX):
        start = jnp.where(
            is0,
            jnp.minimum(i * CH, cnt0 - CH),
            jnp.maximum(T - (i + 1) * CH, cnt0),
        )

        @pl.when(i < n_keep)
        def _(start=start):
            gather_rows(out_ref, start)

    for i in range(N_MAX):

        @pl.when(i < n_send)
        def _(i=i):
            pltpu.make_async_remote_copy(
                src_ref=xs_ref.at[pl.ds(0, CH)],
                dst_ref=out_ref.at[pl.ds(0, CH)],
                send_sem=send_sems.at[i],
                recv_sem=recv_sems.at[i],
                device_id=peer,
                device_id_type=pl.DeviceIdType.MESH,
            ).wait_recv()

    for i in range(N_MAX):

        @pl.when(i < n_send)
        def _(i=i):
            pltpu.make_async_remote_copy(
                src_ref=xs_ref.at[pl.ds(0, CH)],
                dst_ref=out_ref.at[pl.ds(0, CH)],
                send_sem=send_sems.at[i],
                recv_sem=recv_sems.at[i],
                device_id=peer,
                device_id_type=pl.DeviceIdType.MESH,
            ).wait_send()


def kernel(x, dest):
    order = jnp.argsort(dest, stable=True).astype(jnp.int32)
    cnt0 = jnp.sum(dest == 0).astype(jnp.int32).reshape((1,))

    out = pl.pallas_call(
        _body,
        out_shape=jax.ShapeDtypeStruct((T, *ROW), jnp.float32),
        in_specs=[
            pl.BlockSpec(memory_space=pltpu.SMEM),
            pl.BlockSpec(memory_space=pltpu.SMEM),
            pl.BlockSpec(memory_space=pltpu.VMEM),
        ],
        out_specs=pl.BlockSpec(memory_space=pltpu.VMEM),
        scratch_shapes=[
            pltpu.VMEM((T, *ROW), jnp.float32),
            pltpu.SemaphoreType.DMA((N_MAX,)),
            pltpu.SemaphoreType.DMA((N_MAX,)),
        ],
        compiler_params=pltpu.CompilerParams(collective_id=0),
    )(cnt0, order, x)
    return out.reshape(T, D)


# device time: 118308 ns/iter; 1.1012x vs baseline; 1.0149x over previous
import jax
import jax.numpy as jnp
from jax import lax
from jax.experimental import pallas as pl
from jax.experimental.pallas import tpu as pltpu

T = 4096
D = 1024
UNROLL = 8
ROW = (8, 128)
CH = 512
CUM = (0, 128, 256, 512, 1024, 1536, 2048, 2560, 3072, 3584, 4096)
SIZES = tuple(b - a for a, b in zip(CUM[:-1], CUM[1:]))
N_MAX = len(SIZES)


def _body(cnt_ref, order_ref, x_ref, out_ref, xs_ref, send_sems, recv_sems):
    my_x = lax.axis_index("x")
    my_y = lax.axis_index("y")
    my_z = lax.axis_index("z")
    peer = (my_x, my_y, 1 - my_z)
    cnt0 = cnt_ref[0]

    barrier_sem = pltpu.get_barrier_semaphore()
    pl.semaphore_signal(
        barrier_sem, inc=1, device_id=peer,
        device_id_type=pl.DeviceIdType.MESH,
    )
    pl.semaphore_wait(barrier_sem, 1)

    is0 = my_z == 0
    send_count = jnp.where(is0, T - cnt0, cnt0)
    keep_count = T - send_count
    dst_shift = jnp.where(is0, -cnt0, T - cnt0)

    def gather_rows(dst_ref, start, rows):
        def grp(g, _):
            base = start + g * UNROLL
            for u in range(UNROLL):
                j = base + u
                dst_ref[pl.ds(j, 1)] = x_ref[
                    pl.ds(order_ref[j], 1), :
                ].reshape(1, *ROW)
            return 0

        lax.fori_loop(0, rows // UNROLL, grp, 0)

    for i in range(N_MAX):
        sz = SIZES[i]
        src_start = jnp.where(
            is0,
            jnp.maximum(T - CUM[i + 1], cnt0),
            jnp.minimum(CUM[i], cnt0 - sz),
        )
        dst_start = src_start + dst_shift

        @pl.when(CUM[i] < send_count)
        def _(i=i, sz=sz, src_start=src_start, dst_start=dst_start):
            gather_rows(xs_ref, src_start, sz)
            pltpu.make_async_remote_copy(
                src_ref=xs_ref.at[pl.ds(src_start, sz)],
                dst_ref=out_ref.at[pl.ds(dst_start, sz)],
                send_sem=send_sems.at[i],
                recv_sem=recv_sems.at[i],
                device_id=peer,
                device_id_type=pl.DeviceIdType.MESH,
            ).start()

    for i in range(T // CH):
        start = jnp.where(
            is0,
            jnp.minimum(i * CH, cnt0 - CH),
            jnp.maximum(T - (i + 1) * CH, cnt0),
        )

        @pl.when(i * CH < keep_count)
        def _(start=start):
            gather_rows(out_ref, start, CH)

    for i in range(N_MAX):

        @pl.when(CUM[i] < send_count)
        def _(i=i):
            pltpu.make_async_remote_copy(
                src_ref=xs_ref.at[pl.ds(0, SIZES[i])],
                dst_ref=out_ref.at[pl.ds(0, SIZES[i])],
                send_sem=send_sems.at[i],
                recv_sem=recv_sems.at[i],
                device_id=peer,
                device_id_type=pl.DeviceIdType.MESH,
            ).wait_recv()

    for i in range(N_MAX):

        @pl.when(CUM[i] < send_count)
        def _(i=i):
            pltpu.make_async_remote_copy(
                src_ref=xs_ref.at[pl.ds(0, SIZES[i])],
                dst_ref=out_ref.at[pl.ds(0, SIZES[i])],
                send_sem=send_sems.at[i],
                recv_sem=recv_sems.at[i],
                device_id=peer,
                device_id_type=pl.DeviceIdType.MESH,
            ).wait_send()


def kernel(x, dest):
    order = jnp.argsort(dest, stable=True).astype(jnp.int32)
    cnt0 = jnp.sum(dest == 0).astype(jnp.int32).reshape((1,))

    out = pl.pallas_call(
        _body,
        out_shape=jax.ShapeDtypeStruct((T, *ROW), jnp.float32),
        in_specs=[
            pl.BlockSpec(memory_space=pltpu.SMEM),
            pl.BlockSpec(memory_space=pltpu.SMEM),
            pl.BlockSpec(memory_space=pltpu.VMEM),
        ],
        out_specs=pl.BlockSpec(memory_space=pltpu.VMEM),
        scratch_shapes=[
            pltpu.VMEM((T, *ROW), jnp.float32),
            pltpu.SemaphoreType.DMA((N_MAX,)),
            pltpu.SemaphoreType.DMA((N_MAX,)),
        ],
        compiler_params=pltpu.CompilerParams(collective_id=0),
    )(cnt0, order, x)
    return out.reshape(T, D)


# device time: 117532 ns/iter; 1.1085x vs baseline; 1.0066x over previous
import jax
import jax.numpy as jnp
from jax import lax
from jax.experimental import pallas as pl
from jax.experimental.pallas import tpu as pltpu

T = 4096
D = 1024
UNROLL = 8
ROW = (8, 128)
CH = 512
CUM = (0, 128, 256, 512, 1024, 1536, 2048, 2560, 3072, 3584, 4096)
SIZES = tuple(b - a for a, b in zip(CUM[:-1], CUM[1:]))
N_MAX = len(SIZES)


def _body(cnt_ref, order_ref, x_ref, out_ref, xs_ref, send_sems, recv_sems):
    my_x = lax.axis_index("x")
    my_y = lax.axis_index("y")
    my_z = lax.axis_index("z")
    peer = (my_x, my_y, 1 - my_z)
    cnt0 = cnt_ref[0]

    barrier_sem = pltpu.get_barrier_semaphore()
    pl.semaphore_signal(
        barrier_sem, inc=1, device_id=peer,
        device_id_type=pl.DeviceIdType.MESH,
    )
    pl.semaphore_wait(barrier_sem, 1)

    is0 = my_z == 0
    send_count = jnp.where(is0, T - cnt0, cnt0)
    keep_count = T - send_count
    dst_shift = jnp.where(is0, -cnt0, T - cnt0)

    def gather_rows(dst_ref, start, rows):
        def grp(g, _):
            base = start + g * UNROLL
            for u in range(UNROLL):
                j = base + u
                dst_ref[pl.ds(j, 1)] = x_ref[
                    pl.ds(order_ref[j], 1), :
                ].reshape(1, *ROW)
            return 0

        lax.fori_loop(0, rows // UNROLL, grp, 0)

    for i in range(N_MAX):
        sz = SIZES[i]
        src_start = jnp.where(
            is0,
            jnp.maximum(T - CUM[i + 1], cnt0),
            jnp.minimum(CUM[i], cnt0 - sz),
        )
        dst_start = src_start + dst_shift

        @pl.when(CUM[i] < send_count)
        def _(i=i, sz=sz, src_start=src_start, dst_start=dst_start):
            gather_rows(xs_ref, src_start, sz)
            pltpu.make_async_remote_copy(
                src_ref=xs_ref.at[pl.ds(src_start, sz)],
                dst_ref=out_ref.at[pl.ds(dst_start, sz)],
                send_sem=send_sems.at[i],
                recv_sem=recv_sems.at[i],
                device_id=peer,
                device_id_type=pl.DeviceIdType.MESH,
            ).start()

    for i in range(T // CH):
        start = jnp.where(
            is0,
            jnp.minimum(i * CH, cnt0 - CH),
            jnp.maximum(T - (i + 1) * CH, cnt0),
        )

        @pl.when(i * CH < keep_count)
        def _(start=start):
            gather_rows(out_ref, start, CH)

    for i in range(N_MAX):

        @pl.when(CUM[i] < send_count)
        def _(i=i):
            pltpu.make_async_remote_copy(
                src_ref=xs_ref.at[pl.ds(0, SIZES[i])],
                dst_ref=out_ref.at[pl.ds(0, SIZES[i])],
                send_sem=send_sems.at[i],
                recv_sem=recv_sems.at[i],
                device_id=peer,
                device_id_type=pl.DeviceIdType.MESH,
            ).wait_recv()

    for i in range(N_MAX):

        @pl.when(CUM[i] < send_count)
        def _(i=i):
            pltpu.make_async_remote_copy(
                src_ref=xs_ref.at[pl.ds(0, SIZES[i])],
                dst_ref=out_ref.at[pl.ds(0, SIZES[i])],
                send_sem=send_sems.at[i],
                recv_sem=recv_sems.at[i],
                device_id=peer,
                device_id_type=pl.DeviceIdType.MESH,
            ).wait_send()


def kernel(x, dest):
    order = jnp.argsort(dest.astype(jnp.int8), stable=True).astype(jnp.int32)
    cnt0 = jnp.sum(dest == 0).astype(jnp.int32).reshape((1,))

    out = pl.pallas_call(
        _body,
        out_shape=jax.ShapeDtypeStruct((T, *ROW), jnp.float32),
        in_specs=[
            pl.BlockSpec(memory_space=pltpu.SMEM),
            pl.BlockSpec(memory_space=pltpu.SMEM),
            pl.BlockSpec(memory_space=pltpu.VMEM),
        ],
        out_specs=pl.BlockSpec(memory_space=pltpu.VMEM),
        scratch_shapes=[
            pltpu.VMEM((T, *ROW), jnp.float32),
            pltpu.SemaphoreType.DMA((N_MAX,)),
            pltpu.SemaphoreType.DMA((N_MAX,)),
        ],
        compiler_params=pltpu.CompilerParams(collective_id=0),
    )(cnt0, order, x)
    return out.reshape(T, D)
